# Initial kernel scaffold; baseline (speedup 1.0000x reference)
import jax
import jax.numpy as jnp
from jax.experimental import pallas as pl


def kernel(x, edge_index, W1, b1, W2, b2, WL, bL):
    raise NotImplementedError("write your pallas kernel here")


# speedup vs baseline: 15.5981x; 15.5981x over previous
import functools

import jax
import jax.numpy as jnp
from jax import lax
from jax.experimental import pallas as pl
from jax.experimental.pallas import tpu as pltpu
from jax.experimental.pallas import tpu_sc as plsc

_N = 10000
_E = 640000
_ER = _E // 128
_NPT = _N // 16


def _node_split(s):
    start = s * 624 + 8 * jnp.minimum(s, 2)
    cnt = jnp.where(s < 2, 632, 624)
    return start, cnt


def _deg_body(dstR, ones128, zeros16, degp_out, idx_v, ones_v, z16_v, hist_sh):
    c = lax.axis_index("c")
    s = lax.axis_index("s")
    w = c * 16 + s
    start = w * 152 + 8 * jnp.minimum(w, 17)
    cnt = jnp.where(w < 17, 160, 152)

    pltpu.sync_copy(dstR.at[pl.ds(start, 152)], idx_v.at[pl.ds(0, 152)])

    @pl.when(w < 17)
    def _():
        pltpu.sync_copy(dstR.at[pl.ds(start + 152, 8)],
                        idx_v.at[pl.ds(152, 8)])

    nstart, ncnt = _node_split(s)
    pltpu.sync_copy(ones128, ones_v)
    pltpu.sync_copy(zeros16, z16_v)
    for z in range(4):
        pltpu.sync_copy(z16_v.at[pl.ds(0, 156)],
                        hist_sh.at[pl.ds(nstart + z * 156, 156)])

    @pl.when(s < 2)
    def _():
        pltpu.sync_copy(z16_v.at[pl.ds(0, 8)],
                        hist_sh.at[pl.ds(nstart + 624, 8)])

    plsc.subcore_barrier()

    def body(r, carry):
        pltpu.sync_copy(ones_v, hist_sh.at[idx_v.at[r]], add=True)
        return carry

    lax.fori_loop(0, cnt, body, 0)
    plsc.subcore_barrier()
    pltpu.sync_copy(hist_sh.at[pl.ds(nstart, 624)],
                    degp_out.at[c, pl.ds(nstart, 624)])

    @pl.when(s < 2)
    def _():
        pltpu.sync_copy(hist_sh.at[pl.ds(nstart + 624, 8)],
                        degp_out.at[c, pl.ds(nstart + 624, 8)])


@functools.cache
def _deg_call():
    return pl.kernel(
        _deg_body,
        out_type=jax.ShapeDtypeStruct((2, _N, 16), jnp.float32),
        mesh=plsc.VectorSubcoreMesh(core_axis_name="c", subcore_axis_name="s"),
        scratch_types=[
            pltpu.VMEM((160, 128), jnp.int32),
            pltpu.VMEM((128, 16), jnp.float32),
            pltpu.VMEM((156, 16), jnp.float32),
            pltpu.VMEM_SHARED((_N, 16), jnp.float32),
        ],
        compiler_params=pltpu.CompilerParams(use_tc_tiling_on_sc=False),
    )


def _scat_body(P, yflat, srcR, dstR, zeros64, s_out,
               src_v, dst_v, rows_v, y_sh, out_sh):
    c = lax.axis_index("c")
    s = lax.axis_index("s")
    start = s * 312 + 8 * jnp.minimum(s, 1)
    nstart, _ncnt = _node_split(s)

    def edge_chunk(base, size):
        pltpu.sync_copy(srcR.at[pl.ds(base, size)],
                        src_v.at[pl.ds(0, size)])
        pltpu.sync_copy(dstR.at[pl.ds(base, size)],
                        dst_v.at[pl.ds(0, size)])

        def body(r, carry):
            pltpu.sync_copy(y_sh.at[src_v.at[r]], rows_v)
            pltpu.sync_copy(rows_v, out_sh.at[dst_v.at[r]], add=True)
            return carry

        lax.fori_loop(0, size, body, 0)

    for qi in range(P // 2):
        q = c * (P // 2) + qi
        qoff = q * _N

        pltpu.sync_copy(yflat.at[pl.ds(qoff + nstart, 624)],
                        y_sh.at[pl.ds(nstart, 624)])
        pltpu.sync_copy(zeros64.at[pl.ds(0, 128)], rows_v)
        for z in range(4):
            pltpu.sync_copy(rows_v.at[pl.ds(0, 128)],
                            out_sh.at[pl.ds(nstart + z * 128, 128)])
        pltpu.sync_copy(rows_v.at[pl.ds(0, 112)],
                        out_sh.at[pl.ds(nstart + 512, 112)])

        @pl.when(s < 2)
        def _():
            pltpu.sync_copy(yflat.at[pl.ds(qoff + nstart + 624, 8)],
                            y_sh.at[pl.ds(nstart + 624, 8)])
            pltpu.sync_copy(rows_v.at[pl.ds(0, 8)],
                            out_sh.at[pl.ds(nstart + 624, 8)])

        plsc.subcore_barrier()

        def full_chunk(k, carry):
            edge_chunk(start + k * 64, 64)
            return carry

        @pl.when(s < 1)
        def _():
            lax.fori_loop(0, 5, full_chunk, 0)

        @pl.when(s >= 1)
        def _():
            lax.fori_loop(0, 4, full_chunk, 0)
            edge_chunk(start + 256, 56)

        plsc.subcore_barrier()
        pltpu.sync_copy(out_sh.at[pl.ds(nstart, 624)],
                        s_out.at[pl.ds(qoff + nstart, 624)])

        @pl.when(s < 2)
        def _():
            pltpu.sync_copy(out_sh.at[pl.ds(nstart + 624, 8)],
                            s_out.at[pl.ds(qoff + nstart + 624, 8)])


@functools.cache
def _make_scat(P):
    return pl.kernel(
        functools.partial(_scat_body, P),
        out_type=jax.ShapeDtypeStruct((P * _N, 64), jnp.float32),
        mesh=plsc.VectorSubcoreMesh(core_axis_name="c", subcore_axis_name="s"),
        scratch_types=[
            pltpu.VMEM((64, 128), jnp.int32),
            pltpu.VMEM((64, 128), jnp.int32),
            pltpu.VMEM((128, 64), jnp.float32),
            pltpu.VMEM_SHARED((_N, 64), jnp.float32),
            pltpu.VMEM_SHARED((_N, 64), jnp.float32),
        ],
        compiler_params=pltpu.CompilerParams(use_tc_tiling_on_sc=False),
    )


_R = 1000


def _y1_body(x_ref, w1_ref, degp_ref, y1_ref, dis_ref):
    deg = degp_ref[0][:, 0:1] + degp_ref[1][:, 0:1] + 1.0
    dis = lax.rsqrt(deg)
    xw = jnp.dot(x_ref[...], w1_ref[...], preferred_element_type=jnp.float32)
    y = xw * dis
    y1_ref[0] = y[:, :64]
    y1_ref[1] = y[:, 64:]
    dis_ref[...] = dis


def _y1_call(x, W1, degp):
    return pl.pallas_call(
        _y1_body,
        grid=(_N // _R,),
        in_specs=[
            pl.BlockSpec((_R, 128), lambda i: (i, 0)),
            pl.BlockSpec((128, 128), lambda i: (0, 0)),
            pl.BlockSpec((2, _R, 16), lambda i: (0, i, 0)),
        ],
        out_specs=[
            pl.BlockSpec((2, _R, 64), lambda i: (0, i, 0)),
            pl.BlockSpec((_R, 1), lambda i: (i, 0)),
        ],
        out_shape=[
            jax.ShapeDtypeStruct((2, _N, 64), jnp.float32),
            jax.ShapeDtypeStruct((_N, 1), jnp.float32),
        ],
    )(x, W1, degp)


def _mid_body(s1_ref, y1_ref, dis_ref, b1_ref, w2_ref, y2_ref):
    dis = dis_ref[...]
    pre = jnp.concatenate([s1_ref[0] + y1_ref[0], s1_ref[1] + y1_ref[1]],
                          axis=1)
    h = jnp.maximum(pre * dis + b1_ref[...], 0.0)
    xw2 = jnp.dot(h, w2_ref[...], preferred_element_type=jnp.float32)
    y2 = xw2 * dis
    for qq in range(4):
        y2_ref[qq] = y2[:, qq * 64:(qq + 1) * 64]


def _mid_call(s1, y1, dis, b1r, W2):
    return pl.pallas_call(
        _mid_body,
        grid=(_N // _R,),
        in_specs=[
            pl.BlockSpec((2, _R, 64), lambda i: (0, i, 0)),
            pl.BlockSpec((2, _R, 64), lambda i: (0, i, 0)),
            pl.BlockSpec((_R, 1), lambda i: (i, 0)),
            pl.BlockSpec((1, 128), lambda i: (0, 0)),
            pl.BlockSpec((128, 256), lambda i: (0, 0)),
        ],
        out_specs=pl.BlockSpec((4, _R, 64), lambda i: (0, i, 0)),
        out_shape=jax.ShapeDtypeStruct((4, _N, 64), jnp.float32),
    )(s1, y1, dis, b1r, W2)


def _fin_body(s2_ref, y2_ref, dis_ref, b2_ref, wl_ref, bl_ref, out_ref):
    dis = dis_ref[...]
    pre = jnp.concatenate([s2_ref[qq] + y2_ref[qq] for qq in range(4)],
                          axis=1)
    h2 = pre * dis + b2_ref[...]
    out_ref[...] = (jnp.dot(h2, wl_ref[...],
                            preferred_element_type=jnp.float32)
                    + bl_ref[...])


def _fin_call(s2, y2, dis, b2r, WL, bLr):
    return pl.pallas_call(
        _fin_body,
        grid=(_N // _R,),
        in_specs=[
            pl.BlockSpec((4, _R, 64), lambda i: (0, i, 0)),
            pl.BlockSpec((4, _R, 64), lambda i: (0, i, 0)),
            pl.BlockSpec((_R, 1), lambda i: (i, 0)),
            pl.BlockSpec((1, 256), lambda i: (0, 0)),
            pl.BlockSpec((256, 16), lambda i: (0, 0)),
            pl.BlockSpec((1, 16), lambda i: (0, 0)),
        ],
        out_specs=pl.BlockSpec((_R, 16), lambda i: (i, 0)),
        out_shape=jax.ShapeDtypeStruct((_N, 16), jnp.float32),
    )(s2, y2, dis, b2r, WL, bLr)


def kernel(x, edge_index, W1, b1, W2, b2, WL, bL):
    srcR = edge_index[0].reshape(_ER, 128)
    dstR = edge_index[1].reshape(_ER, 128)
    ones128 = jnp.ones((128, 16), jnp.float32)
    zeros16 = jnp.zeros((156, 16), jnp.float32)
    zeros64 = jnp.zeros((156, 64), jnp.float32)

    degp = _deg_call()(dstR, ones128, zeros16)
    y1, dis = _y1_call(x, W1, degp)
    s1 = _make_scat(2)(y1.reshape(2 * _N, 64), srcR, dstR,
                       zeros64).reshape(2, _N, 64)
    y2 = _mid_call(s1, y1, dis, b1.reshape(1, 128), W2)
    s2 = _make_scat(4)(y2.reshape(4 * _N, 64), srcR, dstR,
                       zeros64).reshape(4, _N, 64)
    return _fin_call(s2, y2, dis, b2.reshape(1, 256), WL, bL.reshape(1, 16))
